# baseline (speedup 1.0000x reference)
import jax
import jax.numpy as jnp
from jax.experimental import pallas as pl


def _gat(x, edge_index, W, att_src, att_dst, bias):
    n = x.shape[0]
    src = edge_index[0]
    dst = edge_index[1]
    loop = jnp.arange(n, dtype=src.dtype)
    src = jnp.concatenate([src, loop])
    dst = jnp.concatenate([dst, loop])
    h = x @ W
    a_src = h @ att_src
    a_dst = h @ att_dst
    e = a_src[src] + a_dst[dst]
    e = jnp.where(e > 0, e, 0.2 * e)
    e_max = jax.ops.segment_max(e, dst, num_segments=n)
    e_max = jnp.where(jnp.isfinite(e_max), e_max, 0.0)
    ex = jnp.exp(e - e_max[dst])
    denom = jax.ops.segment_sum(ex, dst, num_segments=n)
    alpha = ex / (denom[dst] + 1e-16)
    out = jax.ops.segment_sum(h[src] * alpha[:, None], dst, num_segments=n)
    return out + bias


def _linear_kernel(x_ref, w_ref, b_ref, o_ref):
    o_ref[...] = jnp.dot(x_ref[...], w_ref[...],
                         preferred_element_type=jnp.float32) + b_ref[...]


def _linear(x, W, b):
    n, d = x.shape
    dout = W.shape[1]
    blk = 1000
    return pl.pallas_call(
        _linear_kernel,
        grid=(n // blk,),
        in_specs=[
            pl.BlockSpec((blk, d), lambda i: (i, 0)),
            pl.BlockSpec((d, dout), lambda i: (0, 0)),
            pl.BlockSpec((1, dout), lambda i: (0, 0)),
        ],
        out_specs=pl.BlockSpec((blk, dout), lambda i: (i, 0)),
        out_shape=jax.ShapeDtypeStruct((n, dout), jnp.float32),
    )(x, W, b.reshape(1, dout))


def kernel(x, edge_index, W1, as1, ad1, b1, W2, as2, ad2, b2, W3, as3, ad3, b3, Wl, bl):
    out = jax.nn.relu(_gat(x, edge_index, W1, as1, ad1, b1))
    out = jax.nn.relu(_gat(out, edge_index, W2, as2, ad2, b2))
    out = jax.nn.relu(_gat(out, edge_index, W3, as3, ad3, b3))
    return _linear(out, Wl, bl)


# speedup vs baseline: 20.6443x; 20.6443x over previous
import dataclasses

import jax
import jax.numpy as jnp
from jax import lax
from jax.experimental import pallas as pl
from jax.experimental.pallas import tpu as pltpu
from jax.experimental.pallas import tpu_sc as plsc

N = 10000
E = 320000
D = 128
NC = 2
NS = 16
DH = D // NC
EPT = E // NS
CHUNK = 128
NCHUNK = (EPT + CHUNK - 1) // CHUNK
EPT_PAD = NCHUNK * CHUNK
N_PAD = 10240
ROWS_PER_TILE = N_PAD // NS
DEN_PAD = N_PAD


def _sc_edge_body(h_hbm, srcx_hbm, dstx_hbm, asrc_hbm, adst_hbm, amax_hbm,
                  u_hbm, den_hbm,
                  asrc_t, adst_t, amax_t, src_t, dst_t, ex_t, rows_t,
                  u_sh, den_sh, sem):
    c = lax.axis_index("c")
    s = lax.axis_index("s")

    pltpu.sync_copy(asrc_hbm, asrc_t)
    pltpu.sync_copy(adst_hbm, adst_t)
    pltpu.sync_copy(amax_hbm, amax_t)
    pltpu.sync_copy(srcx_hbm.at[c, s], src_t)
    pltpu.sync_copy(dstx_hbm.at[s], dst_t)

    @pl.loop(0, CHUNK)
    def _zero_rows(r):
        for q in range(DH // 16):
            rows_t[r, pl.ds(16 * q, 16)] = jnp.zeros((16,), jnp.float32)

    @pl.loop(0, ROWS_PER_TILE // CHUNK)
    def _zero_u(k):
        pltpu.sync_copy(rows_t,
                        u_sh.at[pl.ds(ROWS_PER_TILE * s + CHUNK * k, CHUNK)])

    @pl.loop(0, 5)
    def _zero_den(k):
        pltpu.sync_copy(rows_t.at[0].at[pl.ds(0, 64)],
                        den_sh.at[pl.ds(640 * s + 64 * k + 0, 64)])
        pltpu.sync_copy(rows_t.at[0].at[pl.ds(0, 64)],
                        den_sh.at[pl.ds(640 * s + 64 * k + 320, 64)])

    plsc.subcore_barrier()

    amax_v = amax_t[...]
    coff = c * N

    @pl.loop(0, NCHUNK)
    def _chunk(j):
        for ii in range(CHUNK // 16):
            s16 = src_t[j, pl.ds(16 * ii, 16)] - coff
            d16 = dst_t[j, pl.ds(16 * ii, 16)]
            asv = plsc.load_gather(asrc_t, [s16])
            adv = plsc.load_gather(adst_t, [d16])
            e = asv + adv
            e = jnp.where(e > 0, e, 0.2 * e)
            cm = amax_v + adv
            cm = jnp.where(cm > 0, cm, 0.2 * cm)
            exv = jnp.exp(e - cm)
            lin = j * CHUNK + ii * 16 + lax.iota(jnp.int32, 16)
            exv = jnp.where(lin < EPT, exv, 0.0)
            ex_t[pl.ds(16 * ii, 16)] = exv

        pltpu.sync_copy(ex_t, den_sh.at[dst_t.at[j]], add=True)

        pltpu.async_copy(h_hbm.at[src_t.at[j]], rows_t, sem).wait()

        @pl.loop(0, CHUNK)
        def _scale(r):
            exb = plsc.load_gather(ex_t, [jnp.zeros((16,), jnp.int32) + r])
            for q in range(DH // 16):
                rows_t[r, pl.ds(16 * q, 16)] = \
                    rows_t[r, pl.ds(16 * q, 16)] * exb

        pltpu.sync_copy(rows_t, u_sh.at[dst_t.at[j]], add=True)

    plsc.subcore_barrier()

    pltpu.sync_copy(u_sh.at[pl.ds(ROWS_PER_TILE * s, ROWS_PER_TILE)],
                    u_hbm.at[c, pl.ds(ROWS_PER_TILE * s, ROWS_PER_TILE)])
    pltpu.sync_copy(den_sh.at[pl.ds(640 * s, 640)],
                    den_hbm.at[c, pl.ds(640 * s, 640)])


def _sc_edge_aggregate(h2, srcx2, dstx, a_src, a_dst, amax16):
    mesh = plsc.VectorSubcoreMesh(core_axis_name="c", subcore_axis_name="s",
                                  num_cores=NC, num_subcores=NS)
    cp = pltpu.CompilerParams()
    for fld, val in (("needs_layout_passes", False),
                     ("use_tc_tiling_on_sc", False)):
        if fld in pltpu.CompilerParams.__dataclass_fields__:
            cp = dataclasses.replace(cp, **{fld: val})
    fn = pl.kernel(
        _sc_edge_body,
        out_type=[
            jax.ShapeDtypeStruct((NC, N_PAD, DH), jnp.float32),
            jax.ShapeDtypeStruct((NC, DEN_PAD), jnp.float32),
        ],
        mesh=mesh,
        compiler_params=cp,
        scratch_types=[
            pltpu.VMEM((N,), jnp.float32),
            pltpu.VMEM((N,), jnp.float32),
            pltpu.VMEM((16,), jnp.float32),
            pltpu.VMEM((NCHUNK, CHUNK), jnp.int32),
            pltpu.VMEM((NCHUNK, CHUNK), jnp.int32),
            pltpu.VMEM((CHUNK,), jnp.float32),
            pltpu.VMEM((CHUNK, DH), jnp.float32),
            pltpu.VMEM_SHARED((N_PAD, DH), jnp.float32),
            pltpu.VMEM_SHARED((DEN_PAD,), jnp.float32),
            pltpu.SemaphoreType.DMA,
        ],
    )
    return fn(h2, srcx2, dstx, a_src, a_dst, amax16)


def _pre_body(x_ref, w_ref, a_ref, h_ref, aa_ref):
    h = jnp.dot(x_ref[...], w_ref[...], preferred_element_type=jnp.float32)
    h_ref[...] = h
    aa_ref[...] = jnp.dot(h, a_ref[...], preferred_element_type=jnp.float32)


def _pre(x, W, att):
    blk = 1000
    return pl.pallas_call(
        _pre_body,
        grid=(N // blk,),
        in_specs=[
            pl.BlockSpec((blk, D), lambda i: (i, 0)),
            pl.BlockSpec((D, D), lambda i: (0, 0)),
            pl.BlockSpec((D, 8), lambda i: (0, 0)),
        ],
        out_specs=[
            pl.BlockSpec((blk, D), lambda i: (i, 0)),
            pl.BlockSpec((blk, 8), lambda i: (i, 0)),
        ],
        out_shape=[
            jax.ShapeDtypeStruct((N, D), jnp.float32),
            jax.ShapeDtypeStruct((N, 8), jnp.float32),
        ],
    )(x, W, att)


def _post_body(u_ref, h_ref, exs_ref, dinv_ref, b_ref, o_ref):
    u = jnp.concatenate([u_ref[0], u_ref[1]], axis=-1)
    agg = (u + exs_ref[...] * h_ref[...]) * dinv_ref[...]
    o_ref[...] = jnp.maximum(agg + b_ref[...], 0.0)


def _post(u2, h, exs, dinv, b):
    blk = 1000
    return pl.pallas_call(
        _post_body,
        grid=(N // blk,),
        in_specs=[
            pl.BlockSpec((NC, blk, DH), lambda i: (0, i, 0)),
            pl.BlockSpec((blk, D), lambda i: (i, 0)),
            pl.BlockSpec((blk, 1), lambda i: (i, 0)),
            pl.BlockSpec((blk, 1), lambda i: (i, 0)),
            pl.BlockSpec((1, D), lambda i: (0, 0)),
        ],
        out_specs=pl.BlockSpec((blk, D), lambda i: (i, 0)),
        out_shape=jax.ShapeDtypeStruct((N, D), jnp.float32),
    )(u2, h, exs, dinv, b.reshape(1, D))


def _linear_body(x_ref, w_ref, b_ref, o_ref):
    o_ref[...] = jnp.dot(x_ref[...], w_ref[...],
                         preferred_element_type=jnp.float32) + b_ref[...]


def _linear(x, W, b):
    blk = 1000
    dout = W.shape[1]
    return pl.pallas_call(
        _linear_body,
        grid=(N // blk,),
        in_specs=[
            pl.BlockSpec((blk, D), lambda i: (i, 0)),
            pl.BlockSpec((D, dout), lambda i: (0, 0)),
            pl.BlockSpec((1, dout), lambda i: (0, 0)),
        ],
        out_specs=pl.BlockSpec((blk, dout), lambda i: (i, 0)),
        out_shape=jax.ShapeDtypeStruct((N, dout), jnp.float32),
    )(x, W, b.reshape(1, dout))


def _gat_layer(x, srcx2, dstx, W, att_src, att_dst, bias):
    att = jnp.zeros((D, 8), jnp.float32)
    att = att.at[:, 0].set(att_src).at[:, 1].set(att_dst)
    h, aa = _pre(x, W, att)
    a_src = aa[:, 0]
    a_dst = aa[:, 1]
    amax = jnp.max(a_src)
    cshift = jnp.where(amax + a_dst > 0, amax + a_dst, 0.2 * (amax + a_dst))
    e_self = a_src + a_dst
    e_self = jnp.where(e_self > 0, e_self, 0.2 * e_self)
    exs = jnp.exp(e_self - cshift)
    amax16 = jnp.full((16,), amax, jnp.float32)
    h2 = jnp.concatenate([h[:, :DH], h[:, DH:]], axis=0)
    u2, den2 = _sc_edge_aggregate(h2, srcx2, dstx, a_src, a_dst, amax16)
    den = den2[0, :N] + exs
    dinv = 1.0 / (den + 1e-16)
    return _post(u2, h, exs.reshape(N, 1), dinv.reshape(N, 1), bias)


def kernel(x, edge_index, W1, as1, ad1, b1, W2, as2, ad2, b2, W3, as3, ad3,
           b3, Wl, bl):
    src = edge_index[0].astype(jnp.int32).reshape(NS, EPT)
    dst = edge_index[1].astype(jnp.int32).reshape(NS, EPT)
    pad = EPT_PAD - EPT
    srcx = jnp.pad(src, ((0, 0), (0, pad))).reshape(NS, NCHUNK, CHUNK)
    dstx = jnp.pad(dst, ((0, 0), (0, pad))).reshape(NS, NCHUNK, CHUNK)
    srcx2 = jnp.stack([srcx, srcx + N])

    out = _gat_layer(x, srcx2, dstx, W1, as1, ad1, b1)
    out = _gat_layer(out, srcx2, dstx, W2, as2, ad2, b2)
    out = _gat_layer(out, srcx2, dstx, W3, as3, ad3, b3)
    return _linear(out, Wl, bl)
